# Initial kernel scaffold; baseline (speedup 1.0000x reference)
import jax
import jax.numpy as jnp
from jax.experimental import pallas as pl


def kernel(x, weights):
    raise NotImplementedError("write your pallas kernel here")


# speedup vs baseline: 206.8993x; 206.8993x over previous
import functools

import jax
import jax.numpy as jnp
from jax.experimental import pallas as pl


def _bmu_body(x_ref, wt_ref, o_ref, *, cols):
    q = x_ref[...]
    wt = wt_ref[...]
    w2 = jnp.sum(wt * wt, axis=0)
    dot = jnp.dot(
        q, wt,
        precision=jax.lax.Precision.HIGHEST,
        preferred_element_type=jnp.float32,
    )
    d2 = w2[None, :] - 2.0 * dot
    idx = jnp.argmin(d2, axis=1).astype(jnp.int32)
    o_ref[...] = jnp.stack([idx // cols, idx % cols], axis=1)


def kernel(x, weights):
    rows, cols, d = weights.shape
    b = x.shape[0]
    k = rows * cols
    wt = weights.reshape(k, d).T
    tb = min(b, 256)
    body = functools.partial(_bmu_body, cols=cols)
    return pl.pallas_call(
        body,
        grid=(b // tb,),
        in_specs=[
            pl.BlockSpec((tb, d), lambda i: (i, 0)),
            pl.BlockSpec((d, k), lambda i: (0, 0)),
        ],
        out_specs=pl.BlockSpec((tb, 2), lambda i: (i, 0)),
        out_shape=jax.ShapeDtypeStruct((b, 2), jnp.int32),
    )(x, wt)


# speedup vs baseline: 208.0573x; 1.0056x over previous
import functools

import jax
import jax.numpy as jnp
from jax.experimental import pallas as pl
from jax.experimental.pallas import tpu as pltpu


def _bmu_body(x_ref, wt_ref, o_ref, *, cols):
    q = x_ref[...]
    wt = wt_ref[...]
    w2 = jnp.sum(wt * wt, axis=0)
    dot = jnp.dot(
        q, wt,
        precision=jax.lax.Precision.HIGHEST,
        preferred_element_type=jnp.float32,
    )
    d2 = w2[None, :] - 2.0 * dot
    idx = jnp.argmin(d2, axis=1).astype(jnp.int32)
    o_ref[...] = jnp.stack([idx // cols, idx % cols], axis=1)


def kernel(x, weights):
    rows, cols, d = weights.shape
    b = x.shape[0]
    k = rows * cols
    wt = weights.reshape(k, d).T
    tb = min(b, 256)
    body = functools.partial(_bmu_body, cols=cols)
    return pl.pallas_call(
        body,
        grid=(b // tb,),
        compiler_params=pltpu.CompilerParams(
            dimension_semantics=("parallel",),
        ),
        in_specs=[
            pl.BlockSpec((tb, d), lambda i: (i, 0)),
            pl.BlockSpec((d, k), lambda i: (0, 0)),
        ],
        out_specs=pl.BlockSpec((tb, 2), lambda i: (i, 0)),
        out_shape=jax.ShapeDtypeStruct((b, 2), jnp.int32),
    )(x, wt)


# speedup vs baseline: 208.9764x; 1.0044x over previous
import functools

import jax
import jax.numpy as jnp
from jax.experimental import pallas as pl
from jax.experimental.pallas import tpu as pltpu


def _bmu_body(x_ref, wt_ref, o_ref, *, cols):
    q = x_ref[...]
    wt = wt_ref[...]
    w2 = jnp.sum(wt * wt, axis=0)
    dot = jnp.dot(
        q, wt,
        precision=jax.lax.Precision.HIGHEST,
        preferred_element_type=jnp.float32,
    )
    d2 = w2[None, :] - 2.0 * dot
    idx = jnp.argmin(d2, axis=1).astype(jnp.int32)
    o_ref[...] = jnp.stack([idx // cols, idx % cols], axis=1)


def kernel(x, weights):
    rows, cols, d = weights.shape
    b = x.shape[0]
    k = rows * cols
    wt = weights.reshape(k, d).T
    tb = min(b, 512)
    body = functools.partial(_bmu_body, cols=cols)
    return pl.pallas_call(
        body,
        grid=(b // tb,),
        compiler_params=pltpu.CompilerParams(
            dimension_semantics=("parallel",),
        ),
        in_specs=[
            pl.BlockSpec((tb, d), lambda i: (i, 0)),
            pl.BlockSpec((d, k), lambda i: (0, 0)),
        ],
        out_specs=pl.BlockSpec((tb, 2), lambda i: (i, 0)),
        out_shape=jax.ShapeDtypeStruct((b, 2), jnp.int32),
    )(x, wt)
